# baseline (speedup 1.0000x reference)
import functools

import jax
import jax.numpy as jnp
from jax import lax
from jax.experimental import pallas as pl
from jax.experimental.pallas import tpu as pltpu
from jax.experimental.pallas import tpu_sc as plsc

B, L = 4096, 200
GLOVE, FEAT = 128, 64
D = GLOVE + FEAT
DP = 256
VOCAB = 100000

NC, NS = 2, 16
NW = NC * NS
NSLICE = 8
BS = B // NSLICE
B_PER_W = BS // NW
GA, GB = 104, 96

_mesh = plsc.VectorSubcoreMesh(
    core_axis_name="c", subcore_axis_name="s", num_cores=NC, num_subcores=NS
)


GLROWS = 128
NBLK = (BS * L) // (NW * GLROWS)
NCOMB = 3
NIDX = 6


def _make_sc_gather(slice_idx):
    @functools.partial(
        pl.kernel,
        out_type=jax.ShapeDtypeStruct((BS * L, DP), jnp.float32),
        mesh=_mesh,
        scratch_types=[
            *[pltpu.VMEM((GLROWS,), jnp.int32) for _ in range(NIDX)],
            *[pltpu.VMEM((GLROWS, DP), jnp.float32) for _ in range(NCOMB)],
            *[pltpu.SemaphoreType.DMA for _ in range(NIDX + 2 * NCOMB)],
        ],
    )
    def _sc_gather(x_hbm, tab_hbm, mid_hbm, *scr):
        idxb = scr[0:NIDX]
        comb = scr[NIDX:NIDX + NCOMB]
        si = scr[NIDX + NCOMB:2 * NIDX + NCOMB]
        sg = scr[2 * NIDX + NCOMB:2 * NIDX + 2 * NCOMB]
        sw = scr[2 * NIDX + 2 * NCOMB:]
        wid = lax.axis_index("s") * NC + lax.axis_index("c")
        rbase = wid * (B_PER_W * L)
        xbase = slice_idx * (BS * L) + rbase

        def idx_copy(k):
            return pltpu.make_async_copy(
                x_hbm.at[pl.ds(xbase + k * GLROWS, GLROWS)],
                idxb[k % NIDX],
                si[k % NIDX],
            )

        def gath_copy(k):
            return pltpu.make_async_copy(
                tab_hbm.at[idxb[k % NIDX]], comb[k % NCOMB], sg[k % NCOMB]
            )

        def write_copy(k):
            return pltpu.make_async_copy(
                comb[k % NCOMB],
                mid_hbm.at[pl.ds(rbase + k * GLROWS, GLROWS)],
                sw[k % NCOMB],
            )

        idx_copy(0).start()
        idx_copy(1).start()
        for k in range(NBLK):
            if k >= NCOMB:
                write_copy(k - NCOMB).wait()
            idx_copy(k).wait()
            gath_copy(k).start()
            if k + 2 < NBLK:
                idx_copy(k + 2).start()
            if k >= 2:
                gath_copy(k - 2).wait()
                write_copy(k - 2).start()
        for k in (NBLK - 2, NBLK - 1):
            gath_copy(k).wait()
            write_copy(k).start()
        for k in (NBLK - 3, NBLK - 2, NBLK - 1):
            write_copy(k).wait()

    return _sc_gather


_LB = 8


def _trans_body(mid_ref, o_ref):
    for l in range(_LB):
        t = jnp.transpose(mid_ref[:, l, :], (1, 0))
        o_ref[l] = t[0:D, :]


def _trans_body_carry(mid_ref, carry_ref, o_ref):
    del carry_ref
    _trans_body(mid_ref, o_ref)


def _make_transpose(slice_idx, with_carry):
    out_spec = pl.BlockSpec((_LB, D, BS), lambda l: (l, 0, slice_idx))
    in_specs = [pl.BlockSpec((BS, _LB, DP), lambda l: (0, l, 0))]
    kwargs = {}
    body = _trans_body
    if with_carry:
        in_specs.append(pl.BlockSpec(memory_space=pl.ANY))
        kwargs["input_output_aliases"] = {1: 0}
        body = _trans_body_carry
    return pl.pallas_call(
        body,
        grid=(L // _LB,),
        in_specs=in_specs,
        out_specs=out_spec,
        out_shape=jax.ShapeDtypeStruct((L, D, B), jnp.float32),
        **kwargs,
    )


def _fuse_body(em_ref, of_ref, o_ref):
    o_ref[:, 0:GLOVE] = em_ref[...]
    o_ref[:, GLOVE:D] = of_ref[...]


_FUSE_ROWS = 2000
_fuse_call = pl.pallas_call(
    _fuse_body,
    grid=(VOCAB // _FUSE_ROWS,),
    in_specs=[
        pl.BlockSpec((_FUSE_ROWS, GLOVE), lambda i: (i, 0)),
        pl.BlockSpec((_FUSE_ROWS, FEAT), lambda i: (i, 0)),
    ],
    out_specs=pl.BlockSpec((_FUSE_ROWS, DP), lambda i: (i, 0)),
    out_shape=jax.ShapeDtypeStruct((VOCAB, DP), jnp.float32),
)


def _mask_body(x_ref, o_ref):
    o_ref[...] = x_ref[...] > 0


_mask_call = pl.pallas_call(
    _mask_body,
    out_shape=jax.ShapeDtypeStruct((B, L), jnp.bool_),
)


def kernel(x, em_weight, of_weight):
    xf = x.reshape(-1)
    tab = _fuse_call(em_weight, of_weight)
    out_t = None
    for s in range(NSLICE):
        mid_s = _make_sc_gather(s)(xf, tab).reshape(BS, L, DP)
        if s == 0:
            out_t = _make_transpose(s, with_carry=False)(mid_s)
        else:
            out_t = _make_transpose(s, with_carry=True)(mid_s, out_t)
    out = jnp.transpose(out_t, (2, 0, 1))
    mask = _mask_call(x)
    return out, mask
